# baseline (speedup 1.0000x reference)
import functools

import jax
import jax.numpy as jnp
from jax import lax
from jax.experimental import pallas as pl
from jax.experimental.pallas import tpu as pltpu
from jax.experimental.pallas import tpu_sc as plsc

N = 50000
E = 800000
ND = 128
HD = 64
ED = 16

NC = 2
NS = 16
CG = 128
E_PAD = 32 * 196 * CG
E_H = E_PAD // 2
CH = CG // 2
CHG = 128
CHS = 64
EW = E_PAD // (NC * NS)
ET = E_PAD // NS
NROWS_SP = 50016

_SC_MESH = plsc.VectorSubcoreMesh(core_axis_name="c", subcore_axis_name="s")
_SC_PARAMS = pltpu.CompilerParams(use_tc_tiling_on_sc=False)


def _mm_bias_body(x_ref, w_ref, b_ref, o_ref):
    o_ref[...] = jnp.dot(x_ref[...], w_ref[...],
                         preferred_element_type=jnp.float32) + b_ref[...]


def _tc_mm_bias(x, w, b, br):
    r, k = x.shape
    c = w.shape[1]
    return pl.pallas_call(
        _mm_bias_body,
        grid=(r // br,),
        in_specs=[pl.BlockSpec((br, k), lambda i: (i, 0)),
                  pl.BlockSpec((k, c), lambda i: (0, 0)),
                  pl.BlockSpec((1, c), lambda i: (0, 0))],
        out_specs=pl.BlockSpec((br, c), lambda i: (i, 0)),
        out_shape=jax.ShapeDtypeStruct((r, c), jnp.float32),
    )(x, w, b.reshape(1, c))


def _tables_body(h_ref, w_ref, o_ref):
    h = h_ref[...]
    o_ref[0] = jnp.dot(h, w_ref[0:HD], preferred_element_type=jnp.float32)
    o_ref[1] = jnp.dot(h, w_ref[HD:2 * HD], preferred_element_type=jnp.float32)


def _tc_tables(h, w12, br):
    return pl.pallas_call(
        _tables_body,
        grid=(N // br,),
        in_specs=[pl.BlockSpec((br, HD), lambda i: (i, 0)),
                  pl.BlockSpec((2 * HD, HD), lambda i: (0, 0))],
        out_specs=pl.BlockSpec((2, br, HD), lambda i: (0, i, 0)),
        out_shape=jax.ShapeDtypeStruct((2, N, HD), jnp.float32),
    )(h, w12)


def _edge_mlp_body(pre_ref, efl_ref, efr_ref, wc_ref, b1_ref, w2_ref, b2_ref,
                   o_ref):
    xe = jnp.concatenate(
        [jnp.dot(efl_ref[...], wc_ref[...], preferred_element_type=jnp.float32),
         jnp.dot(efr_ref[...], wc_ref[...], preferred_element_type=jnp.float32)],
        axis=1)
    mi = pre_ref[...] + xe + b1_ref[...]
    o_ref[...] = jnp.dot(jax.nn.gelu(mi), w2_ref[...],
                         preferred_element_type=jnp.float32) + b2_ref[...]


def _tc_edge_mlp(pre2, ef_raw, wc, b1d, w2d, b2d, be):
    nblk = E_H // be
    last = E // be - 1
    return pl.pallas_call(
        _edge_mlp_body,
        grid=(nblk,),
        in_specs=[pl.BlockSpec((be, 2 * HD), lambda i: (i, 0)),
                  pl.BlockSpec((be, ED), lambda i: (i, 0)),
                  pl.BlockSpec((be, ED),
                               lambda i: (jnp.minimum(i + nblk, last), 0)),
                  pl.BlockSpec((ED, HD), lambda i: (0, 0)),
                  pl.BlockSpec((1, 2 * HD), lambda i: (0, 0)),
                  pl.BlockSpec((2 * HD, 2 * HD), lambda i: (0, 0)),
                  pl.BlockSpec((1, 2 * HD), lambda i: (0, 0))],
        out_specs=pl.BlockSpec((be, 2 * HD), lambda i: (i, 0)),
        out_shape=jax.ShapeDtypeStruct((E_H, 2 * HD), jnp.float32),
    )(pre2, ef_raw, ef_raw, wc, b1d.reshape(1, 2 * HD), w2d,
      b2d.reshape(1, 2 * HD))


def _update_body(h_ref, a_ref, w1_ref, b1_ref, w2_ref, b2_ref, o_ref):
    h = h_ref[...]
    ui = (jnp.dot(h, w1_ref[0:HD], preferred_element_type=jnp.float32)
          + jnp.dot(a_ref[...], w1_ref[HD:2 * HD],
                    preferred_element_type=jnp.float32)
          + b1_ref[...])
    o_ref[...] = h + jnp.dot(jax.nn.gelu(ui), w2_ref[...],
                             preferred_element_type=jnp.float32) + b2_ref[...]


def _tc_update(h, agg, w1, b1, w2, b2, br):
    return pl.pallas_call(
        _update_body,
        grid=(N // br,),
        in_specs=[pl.BlockSpec((br, HD), lambda i: (i, 0)),
                  pl.BlockSpec((br, HD), lambda i: (i, 0)),
                  pl.BlockSpec((2 * HD, HD), lambda i: (0, 0)),
                  pl.BlockSpec((1, HD), lambda i: (0, 0)),
                  pl.BlockSpec((HD, HD), lambda i: (0, 0)),
                  pl.BlockSpec((1, HD), lambda i: (0, 0))],
        out_specs=pl.BlockSpec((br, HD), lambda i: (i, 0)),
        out_shape=jax.ShapeDtypeStruct((N, HD), jnp.float32),
    )(h, agg, w1, b1.reshape(1, HD), w2, b2.reshape(1, HD))


def _gather_body(ts_ref, td_ref, se_ref, so_ref, de_ref, do_ref, pre_ref,
                 ise, iso, ide, ido, bse, bso, bde, bdo, wse, wso,
                 sem_i, sem_g, sem_w):
    cid = lax.axis_index("c")
    sid = lax.axis_index("s")
    base = (sid * NC + cid) * (EW // 2)
    nchunks = (EW // 2) // CHG

    idxs = (ise, iso, ide, ido)
    bufs = (bse, bso, bde, bdo)
    srcs = (se_ref, so_ref, de_ref, do_ref)
    tabs = (ts_ref, ts_ref, td_ref, td_ref)
    wbufs = (wse, wso)

    def issue_idx(slot, g):
        p0 = base + g * CHG
        for k in range(4):
            pltpu.async_copy(srcs[k].at[pl.ds(p0, CHG)], idxs[k][slot],
                             sem_i[slot])

    def gather(slot, g):
        p0 = base + g * CHG
        for k in range(4):
            pltpu.make_async_copy(srcs[k].at[pl.ds(p0, CHG)], idxs[k][slot],
                                  sem_i[slot]).wait()
        for k in range(4):
            pltpu.async_copy(tabs[k].at[idxs[k][slot]], bufs[k][slot],
                             sem_g[slot])

    def wo_wait(slot):
        pltpu.make_async_copy(
            wse[slot], pre_ref.at[pl.ds(base, CHG), pl.ds(0, HD)],
            sem_w[slot]).wait()
        pltpu.make_async_copy(
            wso[slot], pre_ref.at[pl.ds(base, CHG), pl.ds(HD, HD)],
            sem_w[slot]).wait()

    def step(slot, g):
        for k in range(4):
            pltpu.make_async_copy(tabs[k].at[idxs[k][slot]], bufs[k][slot],
                                  sem_g[slot]).wait()

        @pl.when(g + 2 < nchunks)
        def _():
            issue_idx(slot, g + 2)

        @pl.when(g >= 2)
        def _():
            wo_wait(slot)

        def addrow(i, c2):
            for k in range(2):
                for c4 in range(HD // 16):
                    sl = pl.ds(c4 * 16, 16)
                    wbufs[k][slot][i, sl] = (bufs[k][slot][i, sl]
                                             + bufs[k + 2][slot][i, sl])
            return c2

        lax.fori_loop(0, CHG, addrow, 0)
        r0 = base + g * CHG
        pltpu.async_copy(wse[slot], pre_ref.at[pl.ds(r0, CHG), pl.ds(0, HD)],
                         sem_w[slot])
        pltpu.async_copy(wso[slot], pre_ref.at[pl.ds(r0, CHG), pl.ds(HD, HD)],
                         sem_w[slot])

        @pl.when(g + 2 < nchunks)
        def _():
            gather(slot, g + 2)

    issue_idx(0, 0)
    gather(0, 0)
    issue_idx(1, 1)
    gather(1, 1)

    def pair(g2, carry):
        g = g2 * 2
        step(0, g)
        step(1, g + 1)
        return carry

    lax.fori_loop(0, nchunks // 2, pair, 0)
    wo_wait(0)
    wo_wait(1)


def _dbuf(shape, dtype):
    return [pltpu.VMEM(shape, dtype), pltpu.VMEM(shape, dtype)]


_sc_gather = pl.kernel(
    _gather_body,
    out_type=jax.ShapeDtypeStruct((E_H, 2 * HD), jnp.float32),
    mesh=_SC_MESH,
    scratch_types=[
        _dbuf((CHG,), jnp.int32),
        _dbuf((CHG,), jnp.int32),
        _dbuf((CHG,), jnp.int32),
        _dbuf((CHG,), jnp.int32),
        _dbuf((CHG, HD), jnp.float32),
        _dbuf((CHG, HD), jnp.float32),
        _dbuf((CHG, HD), jnp.float32),
        _dbuf((CHG, HD), jnp.float32),
        _dbuf((CHG, HD), jnp.float32),
        _dbuf((CHG, HD), jnp.float32),
        [pltpu.SemaphoreType.DMA, pltpu.SemaphoreType.DMA],
        [pltpu.SemaphoreType.DMA, pltpu.SemaphoreType.DMA],
        [pltpu.SemaphoreType.DMA, pltpu.SemaphoreType.DMA],
    ],
    compiler_params=_SC_PARAMS,
)


def _scatter_body(m_ref, dse_ref, dso_ref, agg_ref, ie, io, me, mo, ob, aggs,
                  sem_m, sem_s):
    cid = lax.axis_index("c")
    sid = lax.axis_index("s")
    col0 = cid * (HD // NC)
    hw = HD // NC

    def zrow(i, carry):
        ob[i, pl.ds(0, 16)] = jnp.zeros((16,), jnp.float32)
        ob[i, pl.ds(16, 16)] = jnp.zeros((16,), jnp.float32)
        return carry

    lax.fori_loop(0, 125, zrow, 0)

    def zcopy(j, carry):
        pltpu.sync_copy(ob, aggs.at[pl.ds(sid * 3126 + j * 125, 125)])
        return carry

    lax.fori_loop(0, 25, zcopy, 0)
    pltpu.sync_copy(ob.at[pl.ds(0, 1)], aggs.at[pl.ds(sid * 3126 + 3125, 1)])
    plsc.subcore_barrier()

    base = sid * (ET // 2)
    nchunks = (ET // 2) // CHS

    def issue(slot, g):
        p0 = base + g * CHS
        pltpu.async_copy(dse_ref.at[pl.ds(p0, CHS)], ie[slot], sem_m[slot])
        pltpu.async_copy(dso_ref.at[pl.ds(p0, CHS)], io[slot], sem_m[slot])
        pltpu.async_copy(m_ref.at[pl.ds(p0, CHS), pl.ds(col0, hw)],
                         me[slot], sem_m[slot])
        pltpu.async_copy(m_ref.at[pl.ds(p0, CHS), pl.ds(HD + col0, hw)],
                         mo[slot], sem_m[slot])

    def sadd_wait(slot):
        pltpu.make_async_copy(me[slot], aggs.at[ie[slot]], sem_s[slot]).wait()
        pltpu.make_async_copy(mo[slot], aggs.at[io[slot]], sem_s[slot]).wait()

    def step(slot, g):
        p0 = base + g * CHS
        pltpu.make_async_copy(dse_ref.at[pl.ds(p0, CHS)], ie[slot],
                              sem_m[slot]).wait()
        pltpu.make_async_copy(dso_ref.at[pl.ds(p0, CHS)], io[slot],
                              sem_m[slot]).wait()
        pltpu.make_async_copy(m_ref.at[pl.ds(p0, CHS), pl.ds(col0, hw)],
                              me[slot], sem_m[slot]).wait()
        pltpu.make_async_copy(m_ref.at[pl.ds(p0, CHS), pl.ds(HD + col0, hw)],
                              mo[slot], sem_m[slot]).wait()
        pltpu.async_copy(me[slot], aggs.at[ie[slot]], sem_s[slot], add=True)
        pltpu.async_copy(mo[slot], aggs.at[io[slot]], sem_s[slot], add=True)

        s2 = (slot + 2) % 4

        @pl.when(g >= 2)
        def _():
            sadd_wait(s2)

        @pl.when(g + 2 < nchunks)
        def _():
            issue(s2, g + 2)

    issue(0, 0)
    issue(1, 1)

    def quad(g4, carry):
        g = g4 * 4
        for s in range(4):
            step(s, g + s)
        return carry

    lax.fori_loop(0, nchunks // 4, quad, 0)
    sadd_wait((nchunks - 2) % 4)
    sadd_wait((nchunks - 1) % 4)
    plsc.subcore_barrier()

    def wout(k, carry):
        r0 = sid * 3125 + k * 125
        pltpu.sync_copy(aggs.at[pl.ds(r0, 125)], ob)
        pltpu.sync_copy(ob, agg_ref.at[pl.ds(r0, 125), pl.ds(col0, hw)])
        return carry

    lax.fori_loop(0, 25, wout, 0)


def _qbuf(shape, dtype):
    return [pltpu.VMEM(shape, dtype) for _ in range(4)]


_sc_scatter = pl.kernel(
    _scatter_body,
    out_type=jax.ShapeDtypeStruct((N, HD), jnp.float32),
    mesh=_SC_MESH,
    scratch_types=[
        _qbuf((CHS,), jnp.int32),
        _qbuf((CHS,), jnp.int32),
        _qbuf((CHS, HD // NC), jnp.float32),
        _qbuf((CHS, HD // NC), jnp.float32),
        pltpu.VMEM((125, HD // NC), jnp.float32),
        pltpu.VMEM_SHARED((NROWS_SP, HD // NC), jnp.float32),
        [pltpu.SemaphoreType.DMA for _ in range(4)],
        [pltpu.SemaphoreType.DMA for _ in range(4)],
    ],
    compiler_params=_SC_PARAMS,
)


def kernel(node_features, edge_indices, edge_features, W_in, b_in,
           msg_w1, msg_b1, msg_w2, msg_b2,
           upd_w1, upd_b1, upd_w2, upd_b2, W_out, b_out):
    nf = node_features[0]
    src = edge_indices[0, :, 0].astype(jnp.int32)
    dst = edge_indices[0, :, 1].astype(jnp.int32)
    ef = edge_features[0]

    pad = E_PAD - E
    zpad_i = jnp.zeros((pad,), jnp.int32)
    src_g = jnp.concatenate([src, zpad_i])
    dst_g = jnp.concatenate([dst, zpad_i])
    dst_s = jnp.concatenate([dst, jnp.full((pad,), N, jnp.int32)])
    se, so = src_g[:E_H], src_g[E_H:]
    de, do = dst_g[:E_H], dst_g[E_H:]
    dse, dso = dst_s[:E_H], dst_s[E_H:]

    z = jnp.zeros((HD, HD), jnp.float32)

    h = _tc_mm_bias(nf, W_in, b_in, br=2000)
    for l in range(msg_w1.shape[0]):
        w2d = jnp.concatenate(
            [jnp.concatenate([msg_w2[l], z], 1),
             jnp.concatenate([z, msg_w2[l]], 1)], 0)
        b1d = jnp.concatenate([msg_b1[l], msg_b1[l]])
        b2d = jnp.concatenate([msg_b2[l], msg_b2[l]])

        tables = _tc_tables(h, msg_w1[l][0:2 * HD], br=2000)
        pre2 = _sc_gather(tables[0], tables[1], se, so, de, do)
        m2 = _tc_edge_mlp(pre2, ef, msg_w1[l][2 * HD:], b1d, w2d, b2d,
                          be=256)
        agg = _sc_scatter(m2, dse, dso)
        h = _tc_update(h, agg, upd_w1[l], upd_b1[l], upd_w2[l], upd_b2[l],
                       br=2000)
    out = _tc_mm_bias(h, W_out, b_out, br=2000)
    return out[None]


# speedup vs baseline: 1.8134x; 1.8134x over previous
import functools

import jax
import jax.numpy as jnp
from jax import lax
from jax.experimental import pallas as pl
from jax.experimental.pallas import tpu as pltpu
from jax.experimental.pallas import tpu_sc as plsc

N = 50000
E = 800000
ND = 128
HD = 64
ED = 16

NC = 2
NS = 16
CG = 128
E_PAD = 32 * 196 * CG
E_H = E_PAD // 2
CH = CG // 2
CHG = 128
CHS = 64
EW = E_PAD // (NC * NS)
ET = E_PAD // NS
NROWS_SP = 50016

_SC_MESH = plsc.VectorSubcoreMesh(core_axis_name="c", subcore_axis_name="s")
_SC_PARAMS = pltpu.CompilerParams(use_tc_tiling_on_sc=False)


def _mm_bias_body(x_ref, w_ref, b_ref, o_ref):
    o_ref[...] = jnp.dot(x_ref[...], w_ref[...],
                         preferred_element_type=jnp.float32) + b_ref[...]


def _tc_mm_bias(x, w, b, br):
    r, k = x.shape
    c = w.shape[1]
    return pl.pallas_call(
        _mm_bias_body,
        grid=(r // br,),
        in_specs=[pl.BlockSpec((br, k), lambda i: (i, 0)),
                  pl.BlockSpec((k, c), lambda i: (0, 0)),
                  pl.BlockSpec((1, c), lambda i: (0, 0))],
        out_specs=pl.BlockSpec((br, c), lambda i: (i, 0)),
        out_shape=jax.ShapeDtypeStruct((r, c), jnp.float32),
    )(x, w, b.reshape(1, c))


def _tables_body(h_ref, w_ref, o_ref):
    h = h_ref[...]
    o_ref[0] = jnp.dot(h, w_ref[0:HD], preferred_element_type=jnp.float32)
    o_ref[1] = jnp.dot(h, w_ref[HD:2 * HD], preferred_element_type=jnp.float32)


def _tc_tables(h, w12, br):
    return pl.pallas_call(
        _tables_body,
        grid=(N // br,),
        in_specs=[pl.BlockSpec((br, HD), lambda i: (i, 0)),
                  pl.BlockSpec((2 * HD, HD), lambda i: (0, 0))],
        out_specs=pl.BlockSpec((2, br, HD), lambda i: (0, i, 0)),
        out_shape=jax.ShapeDtypeStruct((2, N, HD), jnp.float32),
    )(h, w12)


def _edge_mlp_body(pre_ref, efl_ref, efr_ref, wc_ref, b1_ref, w2_ref, b2_ref,
                   o_ref):
    xe = jnp.concatenate(
        [jnp.dot(efl_ref[...], wc_ref[...], preferred_element_type=jnp.float32),
         jnp.dot(efr_ref[...], wc_ref[...], preferred_element_type=jnp.float32)],
        axis=1)
    mi = pre_ref[...] + xe + b1_ref[...]
    o_ref[...] = jnp.dot(jax.nn.gelu(mi), w2_ref[...],
                         preferred_element_type=jnp.float32) + b2_ref[...]


def _tc_edge_mlp(pre2, ef_raw, wc, b1d, w2d, b2d, be):
    nblk = E_H // be
    last = (E + be - 1) // be - 1
    return pl.pallas_call(
        _edge_mlp_body,
        grid=(nblk,),
        in_specs=[pl.BlockSpec((be, 2 * HD), lambda i: (i, 0)),
                  pl.BlockSpec((be, ED), lambda i: (i, 0)),
                  pl.BlockSpec((be, ED),
                               lambda i: (jnp.minimum(i + nblk, last), 0)),
                  pl.BlockSpec((ED, HD), lambda i: (0, 0)),
                  pl.BlockSpec((1, 2 * HD), lambda i: (0, 0)),
                  pl.BlockSpec((2 * HD, 2 * HD), lambda i: (0, 0)),
                  pl.BlockSpec((1, 2 * HD), lambda i: (0, 0))],
        out_specs=pl.BlockSpec((be, 2 * HD), lambda i: (i, 0)),
        out_shape=jax.ShapeDtypeStruct((E_H, 2 * HD), jnp.float32),
    )(pre2, ef_raw, ef_raw, wc, b1d.reshape(1, 2 * HD), w2d,
      b2d.reshape(1, 2 * HD))


def _update_body(h_ref, a_ref, w1_ref, b1_ref, w2_ref, b2_ref, o_ref):
    h = h_ref[...]
    ui = (jnp.dot(h, w1_ref[0:HD], preferred_element_type=jnp.float32)
          + jnp.dot(a_ref[...], w1_ref[HD:2 * HD],
                    preferred_element_type=jnp.float32)
          + b1_ref[...])
    o_ref[...] = h + jnp.dot(jax.nn.gelu(ui), w2_ref[...],
                             preferred_element_type=jnp.float32) + b2_ref[...]


def _tc_update(h, agg, w1, b1, w2, b2, br):
    return pl.pallas_call(
        _update_body,
        grid=(N // br,),
        in_specs=[pl.BlockSpec((br, HD), lambda i: (i, 0)),
                  pl.BlockSpec((br, HD), lambda i: (i, 0)),
                  pl.BlockSpec((2 * HD, HD), lambda i: (0, 0)),
                  pl.BlockSpec((1, HD), lambda i: (0, 0)),
                  pl.BlockSpec((HD, HD), lambda i: (0, 0)),
                  pl.BlockSpec((1, HD), lambda i: (0, 0))],
        out_specs=pl.BlockSpec((br, HD), lambda i: (i, 0)),
        out_shape=jax.ShapeDtypeStruct((N, HD), jnp.float32),
    )(h, agg, w1, b1.reshape(1, HD), w2, b2.reshape(1, HD))


def _gather_body(ts_ref, td_ref, se_ref, so_ref, de_ref, do_ref, pre_ref,
                 ise, iso, ide, ido, bse, bso, bde, bdo, wse, wso,
                 sem_i, sem_g, sem_w):
    cid = lax.axis_index("c")
    sid = lax.axis_index("s")
    base = (sid * NC + cid) * (EW // 2)
    nchunks = (EW // 2) // CHG

    idxs = (ise, iso, ide, ido)
    bufs = (bse, bso, bde, bdo)
    srcs = (se_ref, so_ref, de_ref, do_ref)
    tabs = (ts_ref, ts_ref, td_ref, td_ref)
    wbufs = (wse, wso)

    def issue_idx(slot, g):
        p0 = base + g * CHG
        for k in range(4):
            pltpu.async_copy(srcs[k].at[pl.ds(p0, CHG)], idxs[k][slot],
                             sem_i[slot])

    def gather(slot, g):
        p0 = base + g * CHG
        for k in range(4):
            pltpu.make_async_copy(srcs[k].at[pl.ds(p0, CHG)], idxs[k][slot],
                                  sem_i[slot]).wait()
        for k in range(4):
            pltpu.async_copy(tabs[k].at[idxs[k][slot]], bufs[k][slot],
                             sem_g[slot])

    def wo_wait(slot):
        pltpu.make_async_copy(
            wse[slot], pre_ref.at[pl.ds(base, CHG), pl.ds(0, HD)],
            sem_w[slot]).wait()
        pltpu.make_async_copy(
            wso[slot], pre_ref.at[pl.ds(base, CHG), pl.ds(HD, HD)],
            sem_w[slot]).wait()

    def step(slot, g):
        for k in range(4):
            pltpu.make_async_copy(tabs[k].at[idxs[k][slot]], bufs[k][slot],
                                  sem_g[slot]).wait()

        @pl.when(g + 2 < nchunks)
        def _():
            issue_idx(slot, g + 2)

        @pl.when(g >= 2)
        def _():
            wo_wait(slot)

        def addrow(i, c2):
            for k in range(2):
                for c4 in range(HD // 16):
                    sl = pl.ds(c4 * 16, 16)
                    wbufs[k][slot][i, sl] = (bufs[k][slot][i, sl]
                                             + bufs[k + 2][slot][i, sl])
            return c2

        lax.fori_loop(0, CHG, addrow, 0)
        r0 = base + g * CHG
        pltpu.async_copy(wse[slot], pre_ref.at[pl.ds(r0, CHG), pl.ds(0, HD)],
                         sem_w[slot])
        pltpu.async_copy(wso[slot], pre_ref.at[pl.ds(r0, CHG), pl.ds(HD, HD)],
                         sem_w[slot])

        @pl.when(g + 2 < nchunks)
        def _():
            gather(slot, g + 2)

    issue_idx(0, 0)
    gather(0, 0)
    issue_idx(1, 1)
    gather(1, 1)

    def pair(g2, carry):
        g = g2 * 2
        step(0, g)
        step(1, g + 1)
        return carry

    lax.fori_loop(0, nchunks // 2, pair, 0)
    wo_wait(0)
    wo_wait(1)


def _dbuf(shape, dtype):
    return [pltpu.VMEM(shape, dtype), pltpu.VMEM(shape, dtype)]


_sc_gather = pl.kernel(
    _gather_body,
    out_type=jax.ShapeDtypeStruct((E_H, 2 * HD), jnp.float32),
    mesh=_SC_MESH,
    scratch_types=[
        _dbuf((CHG,), jnp.int32),
        _dbuf((CHG,), jnp.int32),
        _dbuf((CHG,), jnp.int32),
        _dbuf((CHG,), jnp.int32),
        _dbuf((CHG, HD), jnp.float32),
        _dbuf((CHG, HD), jnp.float32),
        _dbuf((CHG, HD), jnp.float32),
        _dbuf((CHG, HD), jnp.float32),
        _dbuf((CHG, HD), jnp.float32),
        _dbuf((CHG, HD), jnp.float32),
        [pltpu.SemaphoreType.DMA, pltpu.SemaphoreType.DMA],
        [pltpu.SemaphoreType.DMA, pltpu.SemaphoreType.DMA],
        [pltpu.SemaphoreType.DMA, pltpu.SemaphoreType.DMA],
    ],
    compiler_params=_SC_PARAMS,
)


def _scatter_body(m_ref, dse_ref, dso_ref, agg_ref, ie, io, me, mo, ob, aggs,
                  sem_m, sem_s):
    cid = lax.axis_index("c")
    sid = lax.axis_index("s")
    col0 = cid * (HD // NC)
    hw = HD // NC

    def zrow(i, carry):
        ob[i, pl.ds(0, 16)] = jnp.zeros((16,), jnp.float32)
        ob[i, pl.ds(16, 16)] = jnp.zeros((16,), jnp.float32)
        return carry

    lax.fori_loop(0, 125, zrow, 0)

    def zcopy(j, carry):
        pltpu.sync_copy(ob, aggs.at[pl.ds(sid * 3126 + j * 125, 125)])
        return carry

    lax.fori_loop(0, 25, zcopy, 0)
    pltpu.sync_copy(ob.at[pl.ds(0, 1)], aggs.at[pl.ds(sid * 3126 + 3125, 1)])
    plsc.subcore_barrier()

    base = sid * (ET // 2)
    nchunks = (ET // 2) // CHS

    def issue(slot, g):
        p0 = base + g * CHS
        pltpu.async_copy(dse_ref.at[pl.ds(p0, CHS)], ie[slot], sem_m[slot])
        pltpu.async_copy(dso_ref.at[pl.ds(p0, CHS)], io[slot], sem_m[slot])
        pltpu.async_copy(m_ref.at[pl.ds(p0, CHS), pl.ds(col0, hw)],
                         me[slot], sem_m[slot])
        pltpu.async_copy(m_ref.at[pl.ds(p0, CHS), pl.ds(HD + col0, hw)],
                         mo[slot], sem_m[slot])

    def sadd_wait(slot):
        pltpu.make_async_copy(me[slot], aggs.at[ie[slot]], sem_s[slot]).wait()
        pltpu.make_async_copy(mo[slot], aggs.at[io[slot]], sem_s[slot]).wait()

    def step(slot, g):
        p0 = base + g * CHS
        pltpu.make_async_copy(dse_ref.at[pl.ds(p0, CHS)], ie[slot],
                              sem_m[slot]).wait()
        pltpu.make_async_copy(dso_ref.at[pl.ds(p0, CHS)], io[slot],
                              sem_m[slot]).wait()
        pltpu.make_async_copy(m_ref.at[pl.ds(p0, CHS), pl.ds(col0, hw)],
                              me[slot], sem_m[slot]).wait()
        pltpu.make_async_copy(m_ref.at[pl.ds(p0, CHS), pl.ds(HD + col0, hw)],
                              mo[slot], sem_m[slot]).wait()
        pltpu.async_copy(me[slot], aggs.at[ie[slot]], sem_s[slot], add=True)
        pltpu.async_copy(mo[slot], aggs.at[io[slot]], sem_s[slot], add=True)

        s2 = (slot + 2) % 4

        @pl.when(g >= 2)
        def _():
            sadd_wait(s2)

        @pl.when(g + 2 < nchunks)
        def _():
            issue(s2, g + 2)

    issue(0, 0)
    issue(1, 1)

    def quad(g4, carry):
        g = g4 * 4
        for s in range(4):
            step(s, g + s)
        return carry

    lax.fori_loop(0, nchunks // 4, quad, 0)
    sadd_wait((nchunks - 2) % 4)
    sadd_wait((nchunks - 1) % 4)
    plsc.subcore_barrier()

    def wout(k, carry):
        r0 = sid * 3125 + k * 125
        pltpu.sync_copy(aggs.at[pl.ds(r0, 125)], ob)
        pltpu.sync_copy(ob, agg_ref.at[pl.ds(r0, 125), pl.ds(col0, hw)])
        return carry

    lax.fori_loop(0, 25, wout, 0)


def _qbuf(shape, dtype):
    return [pltpu.VMEM(shape, dtype) for _ in range(4)]


_sc_scatter = pl.kernel(
    _scatter_body,
    out_type=jax.ShapeDtypeStruct((N, HD), jnp.float32),
    mesh=_SC_MESH,
    scratch_types=[
        _qbuf((CHS,), jnp.int32),
        _qbuf((CHS,), jnp.int32),
        _qbuf((CHS, HD // NC), jnp.float32),
        _qbuf((CHS, HD // NC), jnp.float32),
        pltpu.VMEM((125, HD // NC), jnp.float32),
        pltpu.VMEM_SHARED((NROWS_SP, HD // NC), jnp.float32),
        [pltpu.SemaphoreType.DMA for _ in range(4)],
        [pltpu.SemaphoreType.DMA for _ in range(4)],
    ],
    compiler_params=_SC_PARAMS,
)


def kernel(node_features, edge_indices, edge_features, W_in, b_in,
           msg_w1, msg_b1, msg_w2, msg_b2,
           upd_w1, upd_b1, upd_w2, upd_b2, W_out, b_out):
    nf = node_features[0]
    src = edge_indices[0, :, 0].astype(jnp.int32)
    dst = edge_indices[0, :, 1].astype(jnp.int32)
    ef = edge_features[0]

    pad = E_PAD - E
    zpad_i = jnp.zeros((pad,), jnp.int32)
    src_g = jnp.concatenate([src, zpad_i])
    dst_g = jnp.concatenate([dst, zpad_i])
    dst_s = jnp.concatenate([dst, jnp.full((pad,), N, jnp.int32)])
    se, so = src_g[:E_H], src_g[E_H:]
    de, do = dst_g[:E_H], dst_g[E_H:]
    dse, dso = dst_s[:E_H], dst_s[E_H:]

    z = jnp.zeros((HD, HD), jnp.float32)

    h = _tc_mm_bias(nf, W_in, b_in, br=2000)
    for l in range(msg_w1.shape[0]):
        w2d = jnp.concatenate(
            [jnp.concatenate([msg_w2[l], z], 1),
             jnp.concatenate([z, msg_w2[l]], 1)], 0)
        b1d = jnp.concatenate([msg_b1[l], msg_b1[l]])
        b2d = jnp.concatenate([msg_b2[l], msg_b2[l]])

        tables = _tc_tables(h, msg_w1[l][0:2 * HD], br=2000)
        pre2 = _sc_gather(tables[0], tables[1], se, so, de, do)
        m2 = _tc_edge_mlp(pre2, ef, msg_w1[l][2 * HD:], b1d, w2d, b2d,
                          be=2048)
        agg = _sc_scatter(m2, dse, dso)
        h = _tc_update(h, agg, upd_w1[l], upd_b1[l], upd_w2[l], upd_b2[l],
                       br=2000)
    out = _tc_mm_bias(h, W_out, b_out, br=2000)
    return out[None]


# speedup vs baseline: 1.8646x; 1.0282x over previous
import functools

import jax
import jax.numpy as jnp
from jax import lax
from jax.experimental import pallas as pl
from jax.experimental.pallas import tpu as pltpu
from jax.experimental.pallas import tpu_sc as plsc

N = 50000
E = 800000
ND = 128
HD = 64
ED = 16

NC = 2
NS = 16
CG = 128
E_PAD = 32 * 196 * CG
E_H = E_PAD // 2
CH = CG // 2
CHG = 128
CHS = 64
EW = E_PAD // (NC * NS)
ET = E_PAD // NS
NROWS_SP = 50016

_SC_MESH = plsc.VectorSubcoreMesh(core_axis_name="c", subcore_axis_name="s")
_SC_PARAMS = pltpu.CompilerParams(use_tc_tiling_on_sc=False)


def _mm_bias_body(x_ref, w_ref, b_ref, o_ref):
    o_ref[...] = jnp.dot(x_ref[...], w_ref[...],
                         preferred_element_type=jnp.float32) + b_ref[...]


def _tc_mm_bias(x, w, b, br):
    r, k = x.shape
    c = w.shape[1]
    return pl.pallas_call(
        _mm_bias_body,
        grid=(r // br,),
        in_specs=[pl.BlockSpec((br, k), lambda i: (i, 0)),
                  pl.BlockSpec((k, c), lambda i: (0, 0)),
                  pl.BlockSpec((1, c), lambda i: (0, 0))],
        out_specs=pl.BlockSpec((br, c), lambda i: (i, 0)),
        out_shape=jax.ShapeDtypeStruct((r, c), jnp.float32),
    )(x, w, b.reshape(1, c))


def _in_tables_body(x_ref, wi_ref, bi_ref, w_ref, h_ref, t_ref):
    h = jnp.dot(x_ref[...], wi_ref[...],
                preferred_element_type=jnp.float32) + bi_ref[...]
    h_ref[...] = h
    t_ref[0] = jnp.dot(h, w_ref[0:HD], preferred_element_type=jnp.float32)
    t_ref[1] = jnp.dot(h, w_ref[HD:2 * HD], preferred_element_type=jnp.float32)


def _tc_in_tables(nf, wi, bi, w12, br):
    return pl.pallas_call(
        _in_tables_body,
        grid=(N // br,),
        in_specs=[pl.BlockSpec((br, ND), lambda i: (i, 0)),
                  pl.BlockSpec((ND, HD), lambda i: (0, 0)),
                  pl.BlockSpec((1, HD), lambda i: (0, 0)),
                  pl.BlockSpec((2 * HD, HD), lambda i: (0, 0))],
        out_specs=[pl.BlockSpec((br, HD), lambda i: (i, 0)),
                   pl.BlockSpec((2, br, HD), lambda i: (0, i, 0))],
        out_shape=[jax.ShapeDtypeStruct((N, HD), jnp.float32),
                   jax.ShapeDtypeStruct((2, N, HD), jnp.float32)],
    )(nf, wi, bi.reshape(1, HD), w12)


def _edge_mlp_body(pre_ref, efl_ref, efr_ref, wc_ref, b1_ref, w2_ref, b2_ref,
                   o_ref):
    xe = jnp.concatenate(
        [jnp.dot(efl_ref[...], wc_ref[...], preferred_element_type=jnp.float32),
         jnp.dot(efr_ref[...], wc_ref[...], preferred_element_type=jnp.float32)],
        axis=1)
    mi = pre_ref[...] + xe + b1_ref[...]
    o_ref[...] = jnp.dot(jax.nn.gelu(mi), w2_ref[...],
                         preferred_element_type=jnp.float32) + b2_ref[...]


def _tc_edge_mlp(pre2, ef_raw, wc, b1d, w2d, b2d, be):
    nblk = E_H // be
    last = (E + be - 1) // be - 1
    return pl.pallas_call(
        _edge_mlp_body,
        grid=(nblk,),
        in_specs=[pl.BlockSpec((be, 2 * HD), lambda i: (i, 0)),
                  pl.BlockSpec((be, ED), lambda i: (i, 0)),
                  pl.BlockSpec((be, ED),
                               lambda i: (jnp.minimum(i + nblk, last), 0)),
                  pl.BlockSpec((ED, HD), lambda i: (0, 0)),
                  pl.BlockSpec((1, 2 * HD), lambda i: (0, 0)),
                  pl.BlockSpec((2 * HD, 2 * HD), lambda i: (0, 0)),
                  pl.BlockSpec((1, 2 * HD), lambda i: (0, 0))],
        out_specs=pl.BlockSpec((be, 2 * HD), lambda i: (i, 0)),
        out_shape=jax.ShapeDtypeStruct((E_H, 2 * HD), jnp.float32),
    )(pre2, ef_raw, ef_raw, wc, b1d.reshape(1, 2 * HD), w2d,
      b2d.reshape(1, 2 * HD))


def _new_h(h_ref, a_ref, w1_ref, b1_ref, w2_ref, b2_ref):
    h = h_ref[...]
    ui = (jnp.dot(h, w1_ref[0:HD], preferred_element_type=jnp.float32)
          + jnp.dot(a_ref[...], w1_ref[HD:2 * HD],
                    preferred_element_type=jnp.float32)
          + b1_ref[...])
    return h + jnp.dot(jax.nn.gelu(ui), w2_ref[...],
                       preferred_element_type=jnp.float32) + b2_ref[...]


def _update_tables_body(h_ref, a_ref, w1_ref, b1_ref, w2_ref, b2_ref, w_ref,
                        hn_ref, t_ref):
    hn = _new_h(h_ref, a_ref, w1_ref, b1_ref, w2_ref, b2_ref)
    hn_ref[...] = hn
    t_ref[0] = jnp.dot(hn, w_ref[0:HD], preferred_element_type=jnp.float32)
    t_ref[1] = jnp.dot(hn, w_ref[HD:2 * HD],
                       preferred_element_type=jnp.float32)


def _tc_update_tables(h, agg, w1, b1, w2, b2, w12, br):
    return pl.pallas_call(
        _update_tables_body,
        grid=(N // br,),
        in_specs=[pl.BlockSpec((br, HD), lambda i: (i, 0)),
                  pl.BlockSpec((br, HD), lambda i: (i, 0)),
                  pl.BlockSpec((2 * HD, HD), lambda i: (0, 0)),
                  pl.BlockSpec((1, HD), lambda i: (0, 0)),
                  pl.BlockSpec((HD, HD), lambda i: (0, 0)),
                  pl.BlockSpec((1, HD), lambda i: (0, 0)),
                  pl.BlockSpec((2 * HD, HD), lambda i: (0, 0))],
        out_specs=[pl.BlockSpec((br, HD), lambda i: (i, 0)),
                   pl.BlockSpec((2, br, HD), lambda i: (0, i, 0))],
        out_shape=[jax.ShapeDtypeStruct((N, HD), jnp.float32),
                   jax.ShapeDtypeStruct((2, N, HD), jnp.float32)],
    )(h, agg, w1, b1.reshape(1, HD), w2, b2.reshape(1, HD), w12)


def _update_out_body(h_ref, a_ref, w1_ref, b1_ref, w2_ref, b2_ref,
                     wo_ref, bo_ref, o_ref):
    hn = _new_h(h_ref, a_ref, w1_ref, b1_ref, w2_ref, b2_ref)
    o_ref[...] = jnp.dot(hn, wo_ref[...],
                         preferred_element_type=jnp.float32) + bo_ref[...]


def _tc_update_out(h, agg, w1, b1, w2, b2, wo, bo, br):
    return pl.pallas_call(
        _update_out_body,
        grid=(N // br,),
        in_specs=[pl.BlockSpec((br, HD), lambda i: (i, 0)),
                  pl.BlockSpec((br, HD), lambda i: (i, 0)),
                  pl.BlockSpec((2 * HD, HD), lambda i: (0, 0)),
                  pl.BlockSpec((1, HD), lambda i: (0, 0)),
                  pl.BlockSpec((HD, HD), lambda i: (0, 0)),
                  pl.BlockSpec((1, HD), lambda i: (0, 0)),
                  pl.BlockSpec((HD, ND), lambda i: (0, 0)),
                  pl.BlockSpec((1, ND), lambda i: (0, 0))],
        out_specs=pl.BlockSpec((br, ND), lambda i: (i, 0)),
        out_shape=jax.ShapeDtypeStruct((N, ND), jnp.float32),
    )(h, agg, w1, b1.reshape(1, HD), w2, b2.reshape(1, HD), wo,
      bo.reshape(1, ND))


def _gather_body(ts_ref, td_ref, se_ref, so_ref, de_ref, do_ref, pre_ref,
                 ise, iso, ide, ido, bse, bso, bde, bdo, wse, wso,
                 sem_i, sem_g, sem_w):
    cid = lax.axis_index("c")
    sid = lax.axis_index("s")
    base = (sid * NC + cid) * (EW // 2)
    nchunks = (EW // 2) // CHG

    idxs = (ise, iso, ide, ido)
    bufs = (bse, bso, bde, bdo)
    srcs = (se_ref, so_ref, de_ref, do_ref)
    tabs = (ts_ref, ts_ref, td_ref, td_ref)
    wbufs = (wse, wso)

    def issue_idx(slot, g):
        p0 = base + g * CHG
        for k in range(4):
            pltpu.async_copy(srcs[k].at[pl.ds(p0, CHG)], idxs[k][slot],
                             sem_i[slot])

    def gather(slot, g):
        p0 = base + g * CHG
        for k in range(4):
            pltpu.make_async_copy(srcs[k].at[pl.ds(p0, CHG)], idxs[k][slot],
                                  sem_i[slot]).wait()
        for k in range(4):
            pltpu.async_copy(tabs[k].at[idxs[k][slot]], bufs[k][slot],
                             sem_g[slot])

    def wo_wait(slot):
        pltpu.make_async_copy(
            wse[slot], pre_ref.at[pl.ds(base, CHG), pl.ds(0, HD)],
            sem_w[slot]).wait()
        pltpu.make_async_copy(
            wso[slot], pre_ref.at[pl.ds(base, CHG), pl.ds(HD, HD)],
            sem_w[slot]).wait()

    def step(slot, g):
        for k in range(4):
            pltpu.make_async_copy(tabs[k].at[idxs[k][slot]], bufs[k][slot],
                                  sem_g[slot]).wait()

        @pl.when(g + 2 < nchunks)
        def _():
            issue_idx(slot, g + 2)

        @pl.when(g >= 2)
        def _():
            wo_wait(slot)

        def addrow(i, c2):
            for k in range(2):
                for c4 in range(HD // 16):
                    sl = pl.ds(c4 * 16, 16)
                    wbufs[k][slot][i, sl] = (bufs[k][slot][i, sl]
                                             + bufs[k + 2][slot][i, sl])
            return c2

        lax.fori_loop(0, CHG, addrow, 0)
        r0 = base + g * CHG
        pltpu.async_copy(wse[slot], pre_ref.at[pl.ds(r0, CHG), pl.ds(0, HD)],
                         sem_w[slot])
        pltpu.async_copy(wso[slot], pre_ref.at[pl.ds(r0, CHG), pl.ds(HD, HD)],
                         sem_w[slot])

        @pl.when(g + 2 < nchunks)
        def _():
            gather(slot, g + 2)

    issue_idx(0, 0)
    gather(0, 0)
    issue_idx(1, 1)
    gather(1, 1)

    def pair(g2, carry):
        g = g2 * 2
        step(0, g)
        step(1, g + 1)
        return carry

    lax.fori_loop(0, nchunks // 2, pair, 0)
    wo_wait(0)
    wo_wait(1)


def _dbuf(shape, dtype):
    return [pltpu.VMEM(shape, dtype), pltpu.VMEM(shape, dtype)]


_sc_gather = pl.kernel(
    _gather_body,
    out_type=jax.ShapeDtypeStruct((E_H, 2 * HD), jnp.float32),
    mesh=_SC_MESH,
    scratch_types=[
        _dbuf((CHG,), jnp.int32),
        _dbuf((CHG,), jnp.int32),
        _dbuf((CHG,), jnp.int32),
        _dbuf((CHG,), jnp.int32),
        _dbuf((CHG, HD), jnp.float32),
        _dbuf((CHG, HD), jnp.float32),
        _dbuf((CHG, HD), jnp.float32),
        _dbuf((CHG, HD), jnp.float32),
        _dbuf((CHG, HD), jnp.float32),
        _dbuf((CHG, HD), jnp.float32),
        [pltpu.SemaphoreType.DMA, pltpu.SemaphoreType.DMA],
        [pltpu.SemaphoreType.DMA, pltpu.SemaphoreType.DMA],
        [pltpu.SemaphoreType.DMA, pltpu.SemaphoreType.DMA],
    ],
    compiler_params=_SC_PARAMS,
)


def _scatter_body(m_ref, dse_ref, dso_ref, agg_ref, ie, io, me, mo, ob, aggs,
                  sem_m, sem_s):
    cid = lax.axis_index("c")
    sid = lax.axis_index("s")
    col0 = cid * (HD // NC)
    hw = HD // NC

    def zrow(i, carry):
        ob[i, pl.ds(0, 16)] = jnp.zeros((16,), jnp.float32)
        ob[i, pl.ds(16, 16)] = jnp.zeros((16,), jnp.float32)
        return carry

    lax.fori_loop(0, 125, zrow, 0)

    def zcopy(j, carry):
        pltpu.sync_copy(ob, aggs.at[pl.ds(sid * 3126 + j * 125, 125)])
        return carry

    lax.fori_loop(0, 25, zcopy, 0)
    pltpu.sync_copy(ob.at[pl.ds(0, 1)], aggs.at[pl.ds(sid * 3126 + 3125, 1)])
    plsc.subcore_barrier()

    base = sid * (ET // 2)
    nchunks = (ET // 2) // CHS

    def issue(slot, g):
        p0 = base + g * CHS
        pltpu.async_copy(dse_ref.at[pl.ds(p0, CHS)], ie[slot], sem_m[slot])
        pltpu.async_copy(dso_ref.at[pl.ds(p0, CHS)], io[slot], sem_m[slot])
        pltpu.async_copy(m_ref.at[pl.ds(p0, CHS), pl.ds(col0, hw)],
                         me[slot], sem_m[slot])
        pltpu.async_copy(m_ref.at[pl.ds(p0, CHS), pl.ds(HD + col0, hw)],
                         mo[slot], sem_m[slot])

    def sadd_wait(slot):
        pltpu.make_async_copy(me[slot], aggs.at[ie[slot]], sem_s[slot]).wait()
        pltpu.make_async_copy(mo[slot], aggs.at[io[slot]], sem_s[slot]).wait()

    def step(slot, g):
        p0 = base + g * CHS
        pltpu.make_async_copy(dse_ref.at[pl.ds(p0, CHS)], ie[slot],
                              sem_m[slot]).wait()
        pltpu.make_async_copy(dso_ref.at[pl.ds(p0, CHS)], io[slot],
                              sem_m[slot]).wait()
        pltpu.make_async_copy(m_ref.at[pl.ds(p0, CHS), pl.ds(col0, hw)],
                              me[slot], sem_m[slot]).wait()
        pltpu.make_async_copy(m_ref.at[pl.ds(p0, CHS), pl.ds(HD + col0, hw)],
                              mo[slot], sem_m[slot]).wait()
        pltpu.async_copy(me[slot], aggs.at[ie[slot]], sem_s[slot], add=True)
        pltpu.async_copy(mo[slot], aggs.at[io[slot]], sem_s[slot], add=True)

        s2 = (slot + 2) % 4

        @pl.when(g >= 2)
        def _():
            sadd_wait(s2)

        @pl.when(g + 2 < nchunks)
        def _():
            issue(s2, g + 2)

    issue(0, 0)
    issue(1, 1)

    def quad(g4, carry):
        g = g4 * 4
        for s in range(4):
            step(s, g + s)
        return carry

    lax.fori_loop(0, nchunks // 4, quad, 0)
    sadd_wait((nchunks - 2) % 4)
    sadd_wait((nchunks - 1) % 4)
    plsc.subcore_barrier()

    def wout(k, carry):
        r0 = sid * 3125 + k * 125
        pltpu.sync_copy(aggs.at[pl.ds(r0, 125)], ob)
        pltpu.sync_copy(ob, agg_ref.at[pl.ds(r0, 125), pl.ds(col0, hw)])
        return carry

    lax.fori_loop(0, 25, wout, 0)


def _qbuf(shape, dtype):
    return [pltpu.VMEM(shape, dtype) for _ in range(4)]


_sc_scatter = pl.kernel(
    _scatter_body,
    out_type=jax.ShapeDtypeStruct((N, HD), jnp.float32),
    mesh=_SC_MESH,
    scratch_types=[
        _qbuf((CHS,), jnp.int32),
        _qbuf((CHS,), jnp.int32),
        _qbuf((CHS, HD // NC), jnp.float32),
        _qbuf((CHS, HD // NC), jnp.float32),
        pltpu.VMEM((125, HD // NC), jnp.float32),
        pltpu.VMEM_SHARED((NROWS_SP, HD // NC), jnp.float32),
        [pltpu.SemaphoreType.DMA for _ in range(4)],
        [pltpu.SemaphoreType.DMA for _ in range(4)],
    ],
    compiler_params=_SC_PARAMS,
)


def kernel(node_features, edge_indices, edge_features, W_in, b_in,
           msg_w1, msg_b1, msg_w2, msg_b2,
           upd_w1, upd_b1, upd_w2, upd_b2, W_out, b_out):
    nf = node_features[0]
    src = edge_indices[0, :, 0].astype(jnp.int32)
    dst = edge_indices[0, :, 1].astype(jnp.int32)
    ef = edge_features[0]

    pad = E_PAD - E
    zpad_i = jnp.zeros((pad,), jnp.int32)
    src_g = jnp.concatenate([src, zpad_i])
    dst_g = jnp.concatenate([dst, zpad_i])
    dst_s = jnp.concatenate([dst, jnp.full((pad,), N, jnp.int32)])
    se, so = src_g[:E_H], src_g[E_H:]
    de, do = dst_g[:E_H], dst_g[E_H:]
    dse, dso = dst_s[:E_H], dst_s[E_H:]

    z = jnp.zeros((HD, HD), jnp.float32)
    nl = msg_w1.shape[0]

    h, tables = _tc_in_tables(nf, W_in, b_in, msg_w1[0][0:2 * HD], br=2000)
    for l in range(nl):
        w2d = jnp.concatenate(
            [jnp.concatenate([msg_w2[l], z], 1),
             jnp.concatenate([z, msg_w2[l]], 1)], 0)
        b1d = jnp.concatenate([msg_b1[l], msg_b1[l]])
        b2d = jnp.concatenate([msg_b2[l], msg_b2[l]])

        pre2 = _sc_gather(tables[0], tables[1], se, so, de, do)
        m2 = _tc_edge_mlp(pre2, ef, msg_w1[l][2 * HD:], b1d, w2d, b2d,
                          be=2048)
        agg = _sc_scatter(m2, dse, dso)
        if l + 1 < nl:
            h, tables = _tc_update_tables(
                h, agg, upd_w1[l], upd_b1[l], upd_w2[l], upd_b2[l],
                msg_w1[l + 1][0:2 * HD], br=2000)
        else:
            out = _tc_update_out(
                h, agg, upd_w1[l], upd_b1[l], upd_w2[l], upd_b2[l],
                W_out, b_out, br=2000)
    return out[None]


# speedup vs baseline: 2.0939x; 1.1230x over previous
import functools

import jax
import jax.numpy as jnp
from jax import lax
from jax.experimental import pallas as pl
from jax.experimental.pallas import tpu as pltpu
from jax.experimental.pallas import tpu_sc as plsc

N = 50000
E = 800000
ND = 128
HD = 64
ED = 16

NC = 2
NS = 16
CG = 128
E_PAD = 32 * 196 * CG
E_H = E_PAD // 2
CH = CG // 2
CHG = 112
CHS = 64
EH2 = E_PAD // 4
GW = EH2 // 32
EW = E_PAD // (NC * NS)
ET = E_PAD // NS
NROWS_SP = 50016

_SC_MESH = plsc.VectorSubcoreMesh(core_axis_name="c", subcore_axis_name="s")
_SC_PARAMS = pltpu.CompilerParams(use_tc_tiling_on_sc=False)


def _mm_bias_body(x_ref, w_ref, b_ref, o_ref):
    o_ref[...] = jnp.dot(x_ref[...], w_ref[...],
                         preferred_element_type=jnp.float32) + b_ref[...]


def _tc_mm_bias(x, w, b, br):
    r, k = x.shape
    c = w.shape[1]
    return pl.pallas_call(
        _mm_bias_body,
        grid=(r // br,),
        in_specs=[pl.BlockSpec((br, k), lambda i: (i, 0)),
                  pl.BlockSpec((k, c), lambda i: (0, 0)),
                  pl.BlockSpec((1, c), lambda i: (0, 0))],
        out_specs=pl.BlockSpec((br, c), lambda i: (i, 0)),
        out_shape=jax.ShapeDtypeStruct((r, c), jnp.float32),
    )(x, w, b.reshape(1, c))


def _in_tables_body(x_ref, wi_ref, bi_ref, w_ref, h_ref, t_ref):
    h = jnp.dot(x_ref[...], wi_ref[...],
                preferred_element_type=jnp.float32) + bi_ref[...]
    h_ref[...] = h
    t_ref[0] = jnp.dot(h, w_ref[0:HD], preferred_element_type=jnp.float32)
    t_ref[1] = jnp.dot(h, w_ref[HD:2 * HD], preferred_element_type=jnp.float32)


def _tc_in_tables(nf, wi, bi, w12, br):
    return pl.pallas_call(
        _in_tables_body,
        grid=(N // br,),
        in_specs=[pl.BlockSpec((br, ND), lambda i: (i, 0)),
                  pl.BlockSpec((ND, HD), lambda i: (0, 0)),
                  pl.BlockSpec((1, HD), lambda i: (0, 0)),
                  pl.BlockSpec((2 * HD, HD), lambda i: (0, 0))],
        out_specs=[pl.BlockSpec((br, HD), lambda i: (i, 0)),
                   pl.BlockSpec((2, br, HD), lambda i: (0, i, 0))],
        out_shape=[jax.ShapeDtypeStruct((N, HD), jnp.float32),
                   jax.ShapeDtypeStruct((2, N, HD), jnp.float32)],
    )(nf, wi, bi.reshape(1, HD), w12)


def _edge_mlp_body(pre_ref, efl_ref, efr_ref, wc_ref, b1_ref, w2_ref, b2_ref,
                   o_ref):
    xe = jnp.concatenate(
        [jnp.dot(efl_ref[...], wc_ref[...], preferred_element_type=jnp.float32),
         jnp.dot(efr_ref[...], wc_ref[...], preferred_element_type=jnp.float32)],
        axis=1)
    mi = pre_ref[...] + xe + b1_ref[...]
    o_ref[...] = jnp.dot(jax.nn.gelu(mi), w2_ref[...],
                         preferred_element_type=jnp.float32) + b2_ref[...]


def _tc_edge_mlp(pre2, ef_raw, wc, b1d, w2d, b2d, be, half):
    nblk = EH2 // be
    o1 = half * nblk
    o2 = half * nblk + E_H // be
    last = (E + be - 1) // be - 1
    return pl.pallas_call(
        _edge_mlp_body,
        grid=(nblk,),
        in_specs=[pl.BlockSpec((be, 2 * HD), lambda i: (i, 0)),
                  pl.BlockSpec((be, ED),
                               lambda i: (jnp.minimum(i + o1, last), 0)),
                  pl.BlockSpec((be, ED),
                               lambda i: (jnp.minimum(i + o2, last), 0)),
                  pl.BlockSpec((ED, HD), lambda i: (0, 0)),
                  pl.BlockSpec((1, 2 * HD), lambda i: (0, 0)),
                  pl.BlockSpec((2 * HD, 2 * HD), lambda i: (0, 0)),
                  pl.BlockSpec((1, 2 * HD), lambda i: (0, 0))],
        out_specs=pl.BlockSpec((be, 2 * HD), lambda i: (i, 0)),
        out_shape=jax.ShapeDtypeStruct((EH2, 2 * HD), jnp.float32),
    )(pre2, ef_raw, ef_raw, wc, b1d.reshape(1, 2 * HD), w2d,
      b2d.reshape(1, 2 * HD))


def _new_h(h_ref, a_ref, a2_ref, w1_ref, b1_ref, w2_ref, b2_ref):
    h = h_ref[...]
    agg = a_ref[...] + a2_ref[...]
    ui = (jnp.dot(h, w1_ref[0:HD], preferred_element_type=jnp.float32)
          + jnp.dot(agg, w1_ref[HD:2 * HD],
                    preferred_element_type=jnp.float32)
          + b1_ref[...])
    return h + jnp.dot(jax.nn.gelu(ui), w2_ref[...],
                       preferred_element_type=jnp.float32) + b2_ref[...]


def _update_tables_body(h_ref, a_ref, a2_ref, w1_ref, b1_ref, w2_ref, b2_ref,
                        w_ref, hn_ref, t_ref):
    hn = _new_h(h_ref, a_ref, a2_ref, w1_ref, b1_ref, w2_ref, b2_ref)
    hn_ref[...] = hn
    t_ref[0] = jnp.dot(hn, w_ref[0:HD], preferred_element_type=jnp.float32)
    t_ref[1] = jnp.dot(hn, w_ref[HD:2 * HD],
                       preferred_element_type=jnp.float32)


def _tc_update_tables(h, agg, agg2, w1, b1, w2, b2, w12, br):
    return pl.pallas_call(
        _update_tables_body,
        grid=(N // br,),
        in_specs=[pl.BlockSpec((br, HD), lambda i: (i, 0)),
                  pl.BlockSpec((br, HD), lambda i: (i, 0)),
                  pl.BlockSpec((br, HD), lambda i: (i, 0)),
                  pl.BlockSpec((2 * HD, HD), lambda i: (0, 0)),
                  pl.BlockSpec((1, HD), lambda i: (0, 0)),
                  pl.BlockSpec((HD, HD), lambda i: (0, 0)),
                  pl.BlockSpec((1, HD), lambda i: (0, 0)),
                  pl.BlockSpec((2 * HD, HD), lambda i: (0, 0))],
        out_specs=[pl.BlockSpec((br, HD), lambda i: (i, 0)),
                   pl.BlockSpec((2, br, HD), lambda i: (0, i, 0))],
        out_shape=[jax.ShapeDtypeStruct((N, HD), jnp.float32),
                   jax.ShapeDtypeStruct((2, N, HD), jnp.float32)],
    )(h, agg, agg2, w1, b1.reshape(1, HD), w2, b2.reshape(1, HD), w12)


def _update_out_body(h_ref, a_ref, a2_ref, w1_ref, b1_ref, w2_ref, b2_ref,
                     wo_ref, bo_ref, o_ref):
    hn = _new_h(h_ref, a_ref, a2_ref, w1_ref, b1_ref, w2_ref, b2_ref)
    o_ref[...] = jnp.dot(hn, wo_ref[...],
                         preferred_element_type=jnp.float32) + bo_ref[...]


def _tc_update_out(h, agg, agg2, w1, b1, w2, b2, wo, bo, br):
    return pl.pallas_call(
        _update_out_body,
        grid=(N // br,),
        in_specs=[pl.BlockSpec((br, HD), lambda i: (i, 0)),
                  pl.BlockSpec((br, HD), lambda i: (i, 0)),
                  pl.BlockSpec((br, HD), lambda i: (i, 0)),
                  pl.BlockSpec((2 * HD, HD), lambda i: (0, 0)),
                  pl.BlockSpec((1, HD), lambda i: (0, 0)),
                  pl.BlockSpec((HD, HD), lambda i: (0, 0)),
                  pl.BlockSpec((1, HD), lambda i: (0, 0)),
                  pl.BlockSpec((HD, ND), lambda i: (0, 0)),
                  pl.BlockSpec((1, ND), lambda i: (0, 0))],
        out_specs=pl.BlockSpec((br, ND), lambda i: (i, 0)),
        out_shape=jax.ShapeDtypeStruct((N, ND), jnp.float32),
    )(h, agg, agg2, w1, b1.reshape(1, HD), w2, b2.reshape(1, HD), wo,
      bo.reshape(1, ND))


def _gather_body(ts_ref, td_ref, se_ref, so_ref, de_ref, do_ref, pre_ref,
                 ise, iso, ide, ido, bse, bso, bde, bdo, wse, wso,
                 sem_i, sem_g, sem_w):
    cid = lax.axis_index("c")
    sid = lax.axis_index("s")
    base = (sid * NC + cid) * GW
    nchunks = GW // CHG

    idxs = (ise, iso, ide, ido)
    bufs = (bse, bso, bde, bdo)
    srcs = (se_ref, so_ref, de_ref, do_ref)
    tabs = (ts_ref, ts_ref, td_ref, td_ref)
    wbufs = (wse, wso)

    def issue_idx(slot, g):
        p0 = base + g * CHG
        for k in range(4):
            pltpu.async_copy(srcs[k].at[pl.ds(p0, CHG)], idxs[k][slot],
                             sem_i[slot])

    def gather(slot, g):
        p0 = base + g * CHG
        for k in range(4):
            pltpu.make_async_copy(srcs[k].at[pl.ds(p0, CHG)], idxs[k][slot],
                                  sem_i[slot]).wait()
        for k in range(4):
            pltpu.async_copy(tabs[k].at[idxs[k][slot]], bufs[k][slot],
                             sem_g[slot])

    def wo_wait(slot):
        pltpu.make_async_copy(
            wse[slot], pre_ref.at[pl.ds(base, CHG), pl.ds(0, HD)],
            sem_w[slot]).wait()
        pltpu.make_async_copy(
            wso[slot], pre_ref.at[pl.ds(base, CHG), pl.ds(HD, HD)],
            sem_w[slot]).wait()

    def step(slot, g):
        for k in range(4):
            pltpu.make_async_copy(tabs[k].at[idxs[k][slot]], bufs[k][slot],
                                  sem_g[slot]).wait()

        @pl.when(g + 2 < nchunks)
        def _():
            issue_idx(slot, g + 2)

        @pl.when(g >= 2)
        def _():
            wo_wait(slot)

        def addrow(i, c2):
            for k in range(2):
                for c4 in range(HD // 16):
                    sl = pl.ds(c4 * 16, 16)
                    wbufs[k][slot][i, sl] = (bufs[k][slot][i, sl]
                                             + bufs[k + 2][slot][i, sl])
            return c2

        lax.fori_loop(0, CHG, addrow, 0)
        r0 = base + g * CHG
        pltpu.async_copy(wse[slot], pre_ref.at[pl.ds(r0, CHG), pl.ds(0, HD)],
                         sem_w[slot])
        pltpu.async_copy(wso[slot], pre_ref.at[pl.ds(r0, CHG), pl.ds(HD, HD)],
                         sem_w[slot])

        @pl.when(g + 2 < nchunks)
        def _():
            gather(slot, g + 2)

    issue_idx(0, 0)
    gather(0, 0)
    issue_idx(1, 1)
    gather(1, 1)

    def pair(g2, carry):
        g = g2 * 2
        step(0, g)
        step(1, g + 1)
        return carry

    lax.fori_loop(0, nchunks // 2, pair, 0)
    wo_wait(0)
    wo_wait(1)


def _dbuf(shape, dtype):
    return [pltpu.VMEM(shape, dtype), pltpu.VMEM(shape, dtype)]


_sc_gather = pl.kernel(
    _gather_body,
    out_type=jax.ShapeDtypeStruct((EH2, 2 * HD), jnp.float32),
    mesh=_SC_MESH,
    scratch_types=[
        _dbuf((CHG,), jnp.int32),
        _dbuf((CHG,), jnp.int32),
        _dbuf((CHG,), jnp.int32),
        _dbuf((CHG,), jnp.int32),
        _dbuf((CHG, HD), jnp.float32),
        _dbuf((CHG, HD), jnp.float32),
        _dbuf((CHG, HD), jnp.float32),
        _dbuf((CHG, HD), jnp.float32),
        _dbuf((CHG, HD), jnp.float32),
        _dbuf((CHG, HD), jnp.float32),
        [pltpu.SemaphoreType.DMA, pltpu.SemaphoreType.DMA],
        [pltpu.SemaphoreType.DMA, pltpu.SemaphoreType.DMA],
        [pltpu.SemaphoreType.DMA, pltpu.SemaphoreType.DMA],
    ],
    compiler_params=_SC_PARAMS,
)


def _scatter_body(m_ref, dse_ref, dso_ref, agg_ref, ie, io, me, mo, ob, aggs,
                  sem_m, sem_s):
    cid = lax.axis_index("c")
    sid = lax.axis_index("s")
    col0 = cid * (HD // NC)
    hw = HD // NC

    def zrow(i, carry):
        ob[i, pl.ds(0, 16)] = jnp.zeros((16,), jnp.float32)
        ob[i, pl.ds(16, 16)] = jnp.zeros((16,), jnp.float32)
        return carry

    lax.fori_loop(0, 125, zrow, 0)

    def zcopy(j, carry):
        pltpu.sync_copy(ob, aggs.at[pl.ds(sid * 3126 + j * 125, 125)])
        return carry

    lax.fori_loop(0, 25, zcopy, 0)
    pltpu.sync_copy(ob.at[pl.ds(0, 1)], aggs.at[pl.ds(sid * 3126 + 3125, 1)])
    plsc.subcore_barrier()

    base = sid * (EH2 // NS)
    nchunks = (EH2 // NS) // CHS

    def issue(slot, g):
        p0 = base + g * CHS
        pltpu.async_copy(dse_ref.at[pl.ds(p0, CHS)], ie[slot], sem_m[slot])
        pltpu.async_copy(dso_ref.at[pl.ds(p0, CHS)], io[slot], sem_m[slot])
        pltpu.async_copy(m_ref.at[pl.ds(p0, CHS), pl.ds(col0, hw)],
                         me[slot], sem_m[slot])
        pltpu.async_copy(m_ref.at[pl.ds(p0, CHS), pl.ds(HD + col0, hw)],
                         mo[slot], sem_m[slot])

    def sadd_wait(slot):
        pltpu.make_async_copy(me[slot], aggs.at[ie[slot]], sem_s[slot]).wait()
        pltpu.make_async_copy(mo[slot], aggs.at[io[slot]], sem_s[slot]).wait()

    def step(slot, g):
        p0 = base + g * CHS
        pltpu.make_async_copy(dse_ref.at[pl.ds(p0, CHS)], ie[slot],
                              sem_m[slot]).wait()
        pltpu.make_async_copy(dso_ref.at[pl.ds(p0, CHS)], io[slot],
                              sem_m[slot]).wait()
        pltpu.make_async_copy(m_ref.at[pl.ds(p0, CHS), pl.ds(col0, hw)],
                              me[slot], sem_m[slot]).wait()
        pltpu.make_async_copy(m_ref.at[pl.ds(p0, CHS), pl.ds(HD + col0, hw)],
                              mo[slot], sem_m[slot]).wait()
        pltpu.async_copy(me[slot], aggs.at[ie[slot]], sem_s[slot], add=True)
        pltpu.async_copy(mo[slot], aggs.at[io[slot]], sem_s[slot], add=True)

        s2 = (slot + 2) % 4

        @pl.when(g >= 2)
        def _():
            sadd_wait(s2)

        @pl.when(g + 2 < nchunks)
        def _():
            issue(s2, g + 2)

    issue(0, 0)
    issue(1, 1)

    def quad(g4, carry):
        g = g4 * 4
        for s in range(4):
            step(s, g + s)
        return carry

    lax.fori_loop(0, nchunks // 4, quad, 0)
    sadd_wait((nchunks - 2) % 4)
    sadd_wait((nchunks - 1) % 4)
    plsc.subcore_barrier()

    def wout(k, carry):
        r0 = sid * 3125 + k * 125
        pltpu.sync_copy(aggs.at[pl.ds(r0, 125)], ob)
        pltpu.sync_copy(ob, agg_ref.at[pl.ds(r0, 125), pl.ds(col0, hw)])
        return carry

    lax.fori_loop(0, 25, wout, 0)


def _qbuf(shape, dtype):
    return [pltpu.VMEM(shape, dtype) for _ in range(4)]


_sc_scatter = pl.kernel(
    _scatter_body,
    out_type=jax.ShapeDtypeStruct((N, HD), jnp.float32),
    mesh=_SC_MESH,
    scratch_types=[
        _qbuf((CHS,), jnp.int32),
        _qbuf((CHS,), jnp.int32),
        _qbuf((CHS, HD // NC), jnp.float32),
        _qbuf((CHS, HD // NC), jnp.float32),
        pltpu.VMEM((125, HD // NC), jnp.float32),
        pltpu.VMEM_SHARED((NROWS_SP, HD // NC), jnp.float32),
        [pltpu.SemaphoreType.DMA for _ in range(4)],
        [pltpu.SemaphoreType.DMA for _ in range(4)],
    ],
    compiler_params=_SC_PARAMS,
)


def kernel(node_features, edge_indices, edge_features, W_in, b_in,
           msg_w1, msg_b1, msg_w2, msg_b2,
           upd_w1, upd_b1, upd_w2, upd_b2, W_out, b_out):
    nf = node_features[0]
    src = edge_indices[0, :, 0].astype(jnp.int32)
    dst = edge_indices[0, :, 1].astype(jnp.int32)
    ef = edge_features[0]

    pad = E_PAD - E
    zpad_i = jnp.zeros((pad,), jnp.int32)
    src_g = jnp.concatenate([src, zpad_i])
    dst_g = jnp.concatenate([dst, zpad_i])
    dst_s = jnp.concatenate([dst, jnp.full((pad,), N, jnp.int32)])
    se, so = src_g[:E_H], src_g[E_H:]
    de, do = dst_g[:E_H], dst_g[E_H:]
    dse, dso = dst_s[:E_H], dst_s[E_H:]

    z = jnp.zeros((HD, HD), jnp.float32)
    nl = msg_w1.shape[0]

    h, tables = _tc_in_tables(nf, W_in, b_in, msg_w1[0][0:2 * HD], br=2000)
    for l in range(nl):
        w2d = jnp.concatenate(
            [jnp.concatenate([msg_w2[l], z], 1),
             jnp.concatenate([z, msg_w2[l]], 1)], 0)
        b1d = jnp.concatenate([msg_b1[l], msg_b1[l]])
        b2d = jnp.concatenate([msg_b2[l], msg_b2[l]])

        wc = msg_w1[l][2 * HD:]
        pre_a = _sc_gather(tables[0], tables[1], se[:EH2], so[:EH2],
                           de[:EH2], do[:EH2])
        pre_b = _sc_gather(tables[0], tables[1], se[EH2:], so[EH2:],
                           de[EH2:], do[EH2:])
        m_a = _tc_edge_mlp(pre_a, ef, wc, b1d, w2d, b2d, be=2048, half=0)
        m_b = _tc_edge_mlp(pre_b, ef, wc, b1d, w2d, b2d, be=2048, half=1)
        agg_a = _sc_scatter(m_a, dse[:EH2], dso[:EH2])
        agg_b = _sc_scatter(m_b, dse[EH2:], dso[EH2:])
        if l + 1 < nl:
            h, tables = _tc_update_tables(
                h, agg_a, agg_b, upd_w1[l], upd_b1[l], upd_w2[l], upd_b2[l],
                msg_w1[l + 1][0:2 * HD], br=2000)
        else:
            out = _tc_update_out(
                h, agg_a, agg_b, upd_w1[l], upd_b1[l], upd_w2[l], upd_b2[l],
                W_out, b_out, br=2000)
    return out[None]
